# Initial kernel scaffold; baseline (speedup 1.0000x reference)
import jax
import jax.numpy as jnp
from jax.experimental import pallas as pl


def kernel(x, d, char_table, dist_table):
    raise NotImplementedError("write your pallas kernel here")


# speedup vs baseline: 1.9900x; 1.9900x over previous
import functools

import jax
import jax.numpy as jnp
from jax import lax
from jax.experimental import pallas as pl
from jax.experimental.pallas import tpu as pltpu
from jax.experimental.pallas import tpu_sc as plsc

B = 4096
L = 50
CHAR_D = 112
DIST_D = 16
OUT_D = CHAR_D + DIST_D
N_ROWS = B * L
NC = 2
NS = 16
NW = NC * NS
ROWS_PER_W = N_ROWS // NW
G = 128
NCHUNK = ROWS_PER_W // G


def _concat_embed_sc(x_hbm, drep_hbm, char_hbm, dist_hbm, out_hbm,
                     xi_v, di_v, crow_v, drow_v, sem):
    wid = lax.axis_index("s") * NC + lax.axis_index("c")
    base = wid * ROWS_PER_W
    pltpu.sync_copy(x_hbm.at[pl.ds(base, ROWS_PER_W)], xi_v)
    pltpu.sync_copy(drep_hbm.at[pl.ds(base, ROWS_PER_W)], di_v)

    def body(g, carry):
        rowbase = base + g * G
        pltpu.async_copy(char_hbm.at[xi_v.at[pl.ds(g * G, G)]], crow_v, sem).wait()
        pltpu.sync_copy(crow_v, out_hbm.at[pl.ds(rowbase, G), pl.ds(0, CHAR_D)])
        pltpu.async_copy(dist_hbm.at[di_v.at[pl.ds(g * G, G)]], drow_v, sem).wait()
        pltpu.sync_copy(drow_v, out_hbm.at[pl.ds(rowbase, G), pl.ds(CHAR_D, DIST_D)])
        return carry

    lax.fori_loop(0, NCHUNK, body, 0)


@jax.jit
def _run(xf, drepf, char_table, dist_table):
    mesh = plsc.VectorSubcoreMesh(core_axis_name="c", subcore_axis_name="s")
    f = functools.partial(
        pl.kernel,
        mesh=mesh,
        out_type=jax.ShapeDtypeStruct((N_ROWS, OUT_D), jnp.float32),
        scratch_types=[
            pltpu.VMEM((ROWS_PER_W,), jnp.int32),
            pltpu.VMEM((ROWS_PER_W,), jnp.int32),
            pltpu.VMEM((G, CHAR_D), jnp.float32),
            pltpu.VMEM((G, DIST_D), jnp.float32),
            pltpu.SemaphoreType.DMA,
        ],
        compiler_params=pltpu.CompilerParams(use_tc_tiling_on_sc=False),
    )(_concat_embed_sc)
    return f(xf, drepf, char_table, dist_table)


def kernel(x, d, char_table, dist_table):
    xf = x.reshape(N_ROWS)
    drepf = jnp.broadcast_to(d[:, None], (B, L)).reshape(N_ROWS)
    out = _run(xf, drepf, char_table, dist_table)
    return out.reshape(B, L, OUT_D)


# speedup vs baseline: 2.1788x; 1.0949x over previous
import functools

import jax
import jax.numpy as jnp
from jax import lax
from jax.experimental import pallas as pl
from jax.experimental.pallas import tpu as pltpu
from jax.experimental.pallas import tpu_sc as plsc

B = 4096
L = 50
CHAR_D = 112
DIST_D = 16
OUT_D = CHAR_D + DIST_D
N_ROWS = B * L
NC = 2
NS = 16
NW = NC * NS
ROWS_PER_W = N_ROWS // NW
G = 128
NCHUNK = ROWS_PER_W // G
NBUF = 5
PFD = 3
KITER = NCHUNK // NBUF


def _concat_embed_sc(x_hbm, drep_hbm, char_hbm, dist_hbm, out_hbm,
                     xi_v, di_v, *bufs):
    crow = bufs[0:NBUF]
    drow = bufs[NBUF:2 * NBUF]
    cg = bufs[2 * NBUF:3 * NBUF]
    cs = bufs[3 * NBUF:4 * NBUF]
    dg = bufs[4 * NBUF:5 * NBUF]
    ds_ = bufs[5 * NBUF:6 * NBUF]

    wid = lax.axis_index("s") * NC + lax.axis_index("c")
    base = wid * ROWS_PER_W
    pltpu.sync_copy(x_hbm.at[pl.ds(base, ROWS_PER_W)], xi_v)
    pltpu.sync_copy(drep_hbm.at[pl.ds(base, ROWS_PER_W)], di_v)

    def issue_gathers(g, b):
        pltpu.async_copy(char_hbm.at[xi_v.at[pl.ds(g * G, G)]], crow[b], cg[b])
        pltpu.async_copy(dist_hbm.at[di_v.at[pl.ds(g * G, G)]], drow[b], dg[b])

    def wait_gathers(b):
        pltpu.make_async_copy(char_hbm.at[pl.ds(0, G)], crow[b], cg[b]).wait()
        pltpu.make_async_copy(dist_hbm.at[pl.ds(0, G)], drow[b], dg[b]).wait()

    def issue_stores(g, b):
        rowbase = base + g * G
        pltpu.async_copy(crow[b], out_hbm.at[pl.ds(rowbase, G), pl.ds(0, CHAR_D)], cs[b])
        pltpu.async_copy(drow[b], out_hbm.at[pl.ds(rowbase, G), pl.ds(CHAR_D, DIST_D)], ds_[b])

    def wait_stores(b):
        pltpu.make_async_copy(crow[b], out_hbm.at[pl.ds(base, G), pl.ds(0, CHAR_D)], cs[b]).wait()
        pltpu.make_async_copy(drow[b], out_hbm.at[pl.ds(base, G), pl.ds(CHAR_D, DIST_D)], ds_[b]).wait()

    for b in range(PFD):
        issue_gathers(b, b)

    def body(k, carry):
        for b in range(NBUF):
            g = k * NBUF + b
            wait_gathers(b)
            issue_stores(g, b)
            g3 = g + PFD
            b3 = (b + PFD) % NBUF
            if b + PFD < NBUF:
                @pl.when(k >= 1)
                def _():
                    wait_stores(b3)
                    issue_gathers(g3, b3)

                @pl.when(k == 0)
                def _():
                    issue_gathers(g3, b3)
            else:
                @pl.when(k < KITER - 1)
                def _():
                    wait_stores(b3)
                    issue_gathers(g3, b3)
        return carry

    lax.fori_loop(0, KITER, body, 0)

    for b in range(NBUF):
        wait_stores(b)


@jax.jit
def _run(xf, drepf, char_table, dist_table):
    mesh = plsc.VectorSubcoreMesh(core_axis_name="c", subcore_axis_name="s")
    scratch = [
        pltpu.VMEM((ROWS_PER_W,), jnp.int32),
        pltpu.VMEM((ROWS_PER_W,), jnp.int32),
    ]
    scratch += [pltpu.VMEM((G, CHAR_D), jnp.float32) for _ in range(NBUF)]
    scratch += [pltpu.VMEM((G, DIST_D), jnp.float32) for _ in range(NBUF)]
    scratch += [pltpu.SemaphoreType.DMA for _ in range(4 * NBUF)]
    f = functools.partial(
        pl.kernel,
        mesh=mesh,
        out_type=jax.ShapeDtypeStruct((N_ROWS, OUT_D), jnp.float32),
        scratch_types=scratch,
        compiler_params=pltpu.CompilerParams(use_tc_tiling_on_sc=False),
    )(_concat_embed_sc)
    return f(xf, drepf, char_table, dist_table)


def kernel(x, d, char_table, dist_table):
    xf = x.reshape(N_ROWS)
    drepf = jnp.broadcast_to(d[:, None], (B, L)).reshape(N_ROWS)
    out = _run(xf, drepf, char_table, dist_table)
    return out.reshape(B, L, OUT_D)


# speedup vs baseline: 2.5591x; 1.1745x over previous
import functools

import jax
import jax.numpy as jnp
from jax import lax
from jax.experimental import pallas as pl
from jax.experimental.pallas import tpu as pltpu
from jax.experimental.pallas import tpu_sc as plsc

B = 4096
L = 50
CHAR_D = 112
DIST_D = 16
OUT_D = CHAR_D + DIST_D
N_ROWS = B * L
NC = 2
NS = 16
NW = NC * NS
ROWS_PER_W = N_ROWS // NW
BATCH_PER_W = B // NW
G = 128
NCHUNK = ROWS_PER_W // G
NBUF = 5
PFD = 3
KITER = NCHUNK // NBUF


def _concat_embed_sc(x_hbm, d_hbm, char_hbm, dist_hbm, out_hbm,
                     xi_v, dvi_v, dvals_v, *bufs):
    orow = bufs[0:NBUF]
    cg = bufs[NBUF:2 * NBUF]
    cs = bufs[2 * NBUF:3 * NBUF]

    wid = lax.axis_index("s") * NC + lax.axis_index("c")
    base = wid * ROWS_PER_W
    pltpu.sync_copy(x_hbm.at[pl.ds(base, ROWS_PER_W)], xi_v)
    pltpu.sync_copy(d_hbm.at[pl.ds(wid * BATCH_PER_W, BATCH_PER_W)], dvi_v)
    pltpu.async_copy(dist_hbm.at[dvi_v], dvals_v, cg[0]).wait()


    def issue_gather(g, b):
        pltpu.async_copy(char_hbm.at[xi_v.at[pl.ds(g * G, G)]], orow[b], cg[b])

    def wait_gather(b):
        pltpu.make_async_copy(char_hbm.at[pl.ds(0, G)], orow[b], cg[b]).wait()

    def issue_store(g, b):
        pltpu.async_copy(orow[b], out_hbm.at[pl.ds(base + g * G, G)], cs[b])

    def wait_store(b):
        pltpu.make_async_copy(orow[b], out_hbm.at[pl.ds(base, G)], cs[b]).wait()

    def fill_dist(g, b):
        ob = orow[b]

        def fb(i, carry):
            for j in range(4):
                r = i * 4 + j
                lb = (g * G + r) // L
                ob[r, pl.ds(CHAR_D, DIST_D)] = dvals_v[lb, pl.ds(0, DIST_D)]
            return carry

        lax.fori_loop(0, G // 4, fb, 0)

    for b in range(PFD):
        issue_gather(b, b)

    def body(k, carry):
        for b in range(NBUF):
            g = k * NBUF + b
            wait_gather(b)
            fill_dist(g, b)
            issue_store(g, b)
            b3 = (b + PFD) % NBUF
            g3 = g + PFD
            if b + PFD < NBUF:
                @pl.when(k >= 1)
                def _():
                    wait_store(b3)
                    issue_gather(g3, b3)

                @pl.when(k == 0)
                def _():
                    issue_gather(g3, b3)
            else:
                @pl.when(k < KITER - 1)
                def _():
                    wait_store(b3)
                    issue_gather(g3, b3)
        return carry

    lax.fori_loop(0, KITER, body, 0)

    for b in range(NBUF):
        wait_store(b)


@jax.jit
def _run(xf, d, char_table, dist_table):
    mesh = plsc.VectorSubcoreMesh(core_axis_name="c", subcore_axis_name="s")
    scratch = [
        pltpu.VMEM((ROWS_PER_W,), jnp.int32),
        pltpu.VMEM((BATCH_PER_W,), jnp.int32),
        pltpu.VMEM((BATCH_PER_W, OUT_D), jnp.float32),
    ]
    scratch += [pltpu.VMEM((G, OUT_D), jnp.float32) for _ in range(NBUF)]
    scratch += [pltpu.SemaphoreType.DMA for _ in range(2 * NBUF)]
    f = functools.partial(
        pl.kernel,
        mesh=mesh,
        out_type=jax.ShapeDtypeStruct((N_ROWS, OUT_D), jnp.float32),
        scratch_types=scratch,
    )(_concat_embed_sc)
    return f(xf, d, char_table, dist_table)


def kernel(x, d, char_table, dist_table):
    xf = x.reshape(N_ROWS)
    char128 = jnp.pad(char_table, ((0, 0), (0, DIST_D)))
    dist128 = jnp.pad(dist_table, ((0, 0), (0, CHAR_D)))
    out = _run(xf, d, char128, dist128)
    return out.reshape(B, L, OUT_D)
